# baseline (speedup 1.0000x reference)
import functools

import jax
import jax.numpy as jnp
from jax import lax
from jax.experimental import pallas as pl
from jax.experimental.pallas import tpu as pltpu
from jax.experimental.pallas import tpu_sc as plsc

N = 10000
C = 128
E = 320000
NCORES = 2
NSUB = 16
NW = NCORES * NSUB
CHUNK = 128
NCHUNKS = 2560
EPAD = NCHUNKS * CHUNK
CPT0, CB0 = 120, 24
CPT1, CB1 = 40, 8
SLAB = 632
SROWS = SLAB * NSUB
ROWBLK = 1000
SIM_T = 0.3
DEG_T = 20.0


def _prep_body(x_ref, o_ref):
    xb = x_ref[...]
    nrm = jnp.sqrt(jnp.sum(xb * xb, axis=1, keepdims=True))
    o_ref[...] = xb / nrm


_prep = pl.pallas_call(
    _prep_body,
    grid=(N // ROWBLK,),
    in_specs=[pl.BlockSpec((ROWBLK, C), lambda i: (i, 0))],
    out_specs=pl.BlockSpec((ROWBLK, C), lambda i: (i, 0)),
    out_shape=jax.ShapeDtypeStruct((N, C), jnp.float32),
)


_mesh = plsc.VectorSubcoreMesh(core_axis_name="c", subcore_axis_name="s")


@functools.partial(
    pl.kernel,
    out_type=[
        jax.ShapeDtypeStruct((NCORES, SROWS, C), jnp.float32),
        jax.ShapeDtypeStruct((NCORES, NSUB, SROWS), jnp.int32),
    ],
    mesh=_mesh,
    compiler_params=pltpu.CompilerParams(needs_layout_passes=False),
    scratch_types=[
        pltpu.VMEM_SHARED((SROWS, C), jnp.float32),
        pltpu.VMEM((SROWS,), jnp.int32),
        pltpu.VMEM((CB0, CHUNK), jnp.int32),
        pltpu.VMEM((CB0, CHUNK), jnp.int32),
        pltpu.VMEM((CHUNK, C), jnp.float32),
        pltpu.VMEM((CHUNK, C), jnp.float32),
        pltpu.SemaphoreType.DMA,
        pltpu.SemaphoreType.DMA,
        pltpu.SemaphoreType.DMA,
        pltpu.SemaphoreType.DMA,
    ],
)
def _sc_scatter(table, rowi, coli, zrows, zhist, s_out, d_out,
                acc, hist, ridx_b, cidx_b,
                rows0, rows1, gsem0, gsem1, ssem0, ssem1):
    cid = lax.axis_index("c")
    sid = lax.axis_index("s")
    slab = pl.ds(sid * SLAB, SLAB)

    with jax.named_scope("sc_init"):
        pltpu.sync_copy(zrows, acc.at[slab])
        pltpu.sync_copy(zhist, hist)
        plsc.subcore_barrier()

    def gather_start(j, rows, sem):
        pltpu.make_async_copy(table.at[ridx_b.at[j]], rows, sem).start()

    def gather_wait(j, rows, sem):
        pltpu.make_async_copy(table.at[ridx_b.at[j]], rows, sem).wait()

    def scat_start(rows, j, sem):
        pltpu.make_async_copy(rows, acc.at[cidx_b.at[j]], sem).start(add=True)

    def scat_wait(rows, j, sem):
        pltpu.make_async_copy(rows, acc.at[cidx_b.at[j]], sem).wait()

    def count_degrees(j):
        for g in range(CHUNK // 16):
            ix = cidx_b[j, pl.ds(g * 16, 16)]
            cnt, last = plsc.scan_count(ix)
            plsc.addupdate_scatter(hist, [ix], cnt, mask=last)

    def pipe(first_chunk, cpt, cb):
        for b in range(cpt // cb):
            off = first_chunk + b * cb
            pltpu.sync_copy(rowi.at[pl.ds(off, cb)], ridx_b.at[pl.ds(0, cb)])
            pltpu.sync_copy(coli.at[pl.ds(off, cb)], cidx_b.at[pl.ds(0, cb)])
            gather_start(0, rows0, gsem0)
            gather_start(1, rows1, gsem1)

            def body(i, carry):
                k = 2 * i
                gather_wait(k, rows0, gsem0)
                count_degrees(k)
                scat_start(rows0, k, ssem0)
                gather_wait(k + 1, rows1, gsem1)
                count_degrees(k + 1)
                scat_start(rows1, k + 1, ssem1)
                scat_wait(rows0, k, ssem0)
                gather_start(k + 2, rows0, gsem0)
                scat_wait(rows1, k + 1, ssem1)
                gather_start(k + 3, rows1, gsem1)
                return carry

            lax.fori_loop(0, cb // 2 - 1, body, 0)

            k = cb - 2
            gather_wait(k, rows0, gsem0)
            count_degrees(k)
            scat_start(rows0, k, ssem0)
            gather_wait(k + 1, rows1, gsem1)
            count_degrees(k + 1)
            scat_start(rows1, k + 1, ssem1)
            scat_wait(rows0, k, ssem0)
            scat_wait(rows1, k + 1, ssem1)

    with jax.named_scope("sc_edges"):
        @pl.when(cid == 0)
        def _():
            pipe(sid * CPT0, CPT0, CB0)

        @pl.when(cid == 1)
        def _():
            pipe(NSUB * CPT0 + sid * CPT1, CPT1, CB1)

    with jax.named_scope("sc_barrier"):
        plsc.subcore_barrier()

    with jax.named_scope("sc_writeback"):
        pltpu.sync_copy(acc.at[slab], s_out.at[cid, slab])
        pltpu.sync_copy(hist, d_out.at[cid, sid])


def _fin_body(x_ref, s_ref, d_ref, ps_ref, pd_ref, o_ref):
    xb = x_ref[...]
    sv = s_ref[0] + s_ref[1]
    deg = jnp.sum(d_ref[...], axis=1).astype(jnp.float32)
    nrm = jnp.sqrt(jnp.sum(xb * xb, axis=1, keepdims=True))
    xn = xb / nrm
    cdot = jnp.sum(xn * sv, axis=1)
    csim = cdot / jnp.maximum(deg, 1.0)
    sim_m = (csim <= SIM_T) & (deg > 0.0)
    deg_m = deg <= DEG_T
    ps = ps_ref[...]
    pd = pd_ref[...]
    a_ok = jnp.all(ps != 0.0)
    b_ok = jnp.all(pd != 0.0)
    plen = (jnp.where(sim_m & a_ok, 1.0, 0.0)
            + jnp.where(deg_m & b_ok, 1.0, 0.0))
    summed = (jnp.where(sim_m, 1.0, 0.0)[:, None] * ps
              + jnp.where(deg_m, 1.0, 0.0)[:, None] * pd)
    final = jnp.where(plen[:, None] != 0.0,
                      summed / jnp.maximum(plen, 1.0)[:, None], 0.0)
    o_ref[...] = xb + final


_fin = pl.pallas_call(
    _fin_body,
    grid=(N // ROWBLK,),
    in_specs=[
        pl.BlockSpec((ROWBLK, C), lambda i: (i, 0)),
        pl.BlockSpec((NCORES, ROWBLK, C), lambda i: (0, i, 0)),
        pl.BlockSpec((ROWBLK, NW), lambda i: (i, 0)),
        pl.BlockSpec((1, C), lambda i: (0, 0)),
        pl.BlockSpec((1, C), lambda i: (0, 0)),
    ],
    out_specs=pl.BlockSpec((ROWBLK, C), lambda i: (i, 0)),
    out_shape=jax.ShapeDtypeStruct((N, C), jnp.float32),
)


def kernel(x, edge_index, prompt_sim_pt, prompt_degree_pt):
    row = edge_index[0].astype(jnp.int32)
    col = edge_index[1].astype(jnp.int32)
    pad = EPAD - E
    rowp = jnp.concatenate([row, jnp.zeros((pad,), jnp.int32)])
    colp = jnp.concatenate([col, jnp.full((pad,), N, jnp.int32)])
    zrows = jnp.zeros((SLAB, C), jnp.float32)
    zhist = jnp.zeros((SROWS,), jnp.int32)
    table = _prep(x)
    s, d = _sc_scatter(table, rowp.reshape(EPAD // CHUNK, CHUNK),
                       colp.reshape(EPAD // CHUNK, CHUNK), zrows, zhist)
    return _fin(x, s, d.reshape(NW, SROWS).T, prompt_sim_pt, prompt_degree_pt)


# speedup vs baseline: 1.1563x; 1.1563x over previous
import functools

import jax
import jax.numpy as jnp
from jax import lax
from jax.experimental import pallas as pl
from jax.experimental.pallas import tpu as pltpu
from jax.experimental.pallas import tpu_sc as plsc

N = 10000
C = 128
E = 320000
NCORES = 2
NSUB = 16
NW = NCORES * NSUB
CHUNK = 128
NCHUNKS = 2560
EPAD = NCHUNKS * CHUNK
CPT0, CB0 = 144, 24
CPT1, CB1 = 16, 8
SLAB = 632
SROWS = SLAB * NSUB
ROWBLK = 1000
SIM_T = 0.3
DEG_T = 20.0


def _prep_body(x_ref, o_ref):
    xb = x_ref[...]
    nrm = jnp.sqrt(jnp.sum(xb * xb, axis=1, keepdims=True))
    o_ref[...] = xb / nrm


_prep = pl.pallas_call(
    _prep_body,
    grid=(N // ROWBLK,),
    in_specs=[pl.BlockSpec((ROWBLK, C), lambda i: (i, 0))],
    out_specs=pl.BlockSpec((ROWBLK, C), lambda i: (i, 0)),
    out_shape=jax.ShapeDtypeStruct((N, C), jnp.float32),
)


_mesh = plsc.VectorSubcoreMesh(core_axis_name="c", subcore_axis_name="s")


@functools.partial(
    pl.kernel,
    out_type=[
        jax.ShapeDtypeStruct((NCORES, SROWS, C), jnp.float32),
        jax.ShapeDtypeStruct((NCORES, NSUB, SROWS), jnp.int32),
    ],
    mesh=_mesh,
    compiler_params=pltpu.CompilerParams(needs_layout_passes=False),
    scratch_types=[
        pltpu.VMEM_SHARED((SROWS, C), jnp.float32),
        pltpu.VMEM((SROWS,), jnp.int32),
        pltpu.VMEM((CB0, CHUNK), jnp.int32),
        pltpu.VMEM((CB0, CHUNK), jnp.int32),
        pltpu.VMEM((CHUNK, C), jnp.float32),
        pltpu.VMEM((CHUNK, C), jnp.float32),
        pltpu.SemaphoreType.DMA,
        pltpu.SemaphoreType.DMA,
        pltpu.SemaphoreType.DMA,
        pltpu.SemaphoreType.DMA,
    ],
)
def _sc_scatter(table, rowi, coli, zrows, zhist, s_out, d_out,
                acc, hist, ridx_b, cidx_b,
                rows0, rows1, gsem0, gsem1, ssem0, ssem1):
    cid = lax.axis_index("c")
    sid = lax.axis_index("s")
    slab = pl.ds(sid * SLAB, SLAB)

    pltpu.sync_copy(zrows, acc.at[slab])
    pltpu.sync_copy(zhist, hist)
    plsc.subcore_barrier()

    def gather_start(j, rows, sem):
        pltpu.make_async_copy(table.at[ridx_b.at[j]], rows, sem).start()

    def gather_wait(j, rows, sem):
        pltpu.make_async_copy(table.at[ridx_b.at[j]], rows, sem).wait()

    def scat_start(rows, j, sem):
        pltpu.make_async_copy(rows, acc.at[cidx_b.at[j]], sem).start(add=True)

    def scat_wait(rows, j, sem):
        pltpu.make_async_copy(rows, acc.at[cidx_b.at[j]], sem).wait()

    def count_degrees(j):
        for g in range(CHUNK // 16):
            ix = cidx_b[j, pl.ds(g * 16, 16)]
            cnt, last = plsc.scan_count(ix)
            plsc.addupdate_scatter(hist, [ix], cnt, mask=last)

    def pipe(first_chunk, cpt, cb):
        for b in range(cpt // cb):
            off = first_chunk + b * cb
            pltpu.sync_copy(rowi.at[pl.ds(off, cb)], ridx_b.at[pl.ds(0, cb)])
            pltpu.sync_copy(coli.at[pl.ds(off, cb)], cidx_b.at[pl.ds(0, cb)])
            gather_start(0, rows0, gsem0)
            gather_start(1, rows1, gsem1)

            def body(i, carry):
                k = 2 * i
                gather_wait(k, rows0, gsem0)
                count_degrees(k)
                scat_start(rows0, k, ssem0)
                gather_wait(k + 1, rows1, gsem1)
                count_degrees(k + 1)
                scat_start(rows1, k + 1, ssem1)
                scat_wait(rows0, k, ssem0)
                gather_start(k + 2, rows0, gsem0)
                scat_wait(rows1, k + 1, ssem1)
                gather_start(k + 3, rows1, gsem1)
                return carry

            lax.fori_loop(0, cb // 2 - 1, body, 0)

            k = cb - 2
            gather_wait(k, rows0, gsem0)
            count_degrees(k)
            scat_start(rows0, k, ssem0)
            gather_wait(k + 1, rows1, gsem1)
            count_degrees(k + 1)
            scat_start(rows1, k + 1, ssem1)
            scat_wait(rows0, k, ssem0)
            scat_wait(rows1, k + 1, ssem1)

    @pl.when(cid == 0)
    def _():
        pipe(sid * CPT0, CPT0, CB0)

    @pl.when(cid == 1)
    def _():
        pipe(NSUB * CPT0 + sid * CPT1, CPT1, CB1)

    plsc.subcore_barrier()
    pltpu.sync_copy(acc.at[slab], s_out.at[cid, slab])
    pltpu.sync_copy(hist, d_out.at[cid, sid])


def _fin_body(x_ref, s_ref, d_ref, ps_ref, pd_ref, o_ref):
    xb = x_ref[...]
    sv = s_ref[0] + s_ref[1]
    deg = jnp.sum(d_ref[...], axis=1).astype(jnp.float32)
    nrm = jnp.sqrt(jnp.sum(xb * xb, axis=1, keepdims=True))
    xn = xb / nrm
    cdot = jnp.sum(xn * sv, axis=1)
    csim = cdot / jnp.maximum(deg, 1.0)
    sim_m = (csim <= SIM_T) & (deg > 0.0)
    deg_m = deg <= DEG_T
    ps = ps_ref[...]
    pd = pd_ref[...]
    a_ok = jnp.all(ps != 0.0)
    b_ok = jnp.all(pd != 0.0)
    plen = (jnp.where(sim_m & a_ok, 1.0, 0.0)
            + jnp.where(deg_m & b_ok, 1.0, 0.0))
    summed = (jnp.where(sim_m, 1.0, 0.0)[:, None] * ps
              + jnp.where(deg_m, 1.0, 0.0)[:, None] * pd)
    final = jnp.where(plen[:, None] != 0.0,
                      summed / jnp.maximum(plen, 1.0)[:, None], 0.0)
    o_ref[...] = xb + final


_fin = pl.pallas_call(
    _fin_body,
    grid=(N // ROWBLK,),
    in_specs=[
        pl.BlockSpec((ROWBLK, C), lambda i: (i, 0)),
        pl.BlockSpec((NCORES, ROWBLK, C), lambda i: (0, i, 0)),
        pl.BlockSpec((ROWBLK, NW), lambda i: (i, 0)),
        pl.BlockSpec((1, C), lambda i: (0, 0)),
        pl.BlockSpec((1, C), lambda i: (0, 0)),
    ],
    out_specs=pl.BlockSpec((ROWBLK, C), lambda i: (i, 0)),
    out_shape=jax.ShapeDtypeStruct((N, C), jnp.float32),
)


def kernel(x, edge_index, prompt_sim_pt, prompt_degree_pt):
    row = edge_index[0].astype(jnp.int32)
    col = edge_index[1].astype(jnp.int32)
    pad = EPAD - E
    rowp = jnp.concatenate([row, jnp.zeros((pad,), jnp.int32)])
    colp = jnp.concatenate([col, jnp.full((pad,), N, jnp.int32)])
    zrows = jnp.zeros((SLAB, C), jnp.float32)
    zhist = jnp.zeros((SROWS,), jnp.int32)
    table = _prep(x)
    s, d = _sc_scatter(table, rowp.reshape(EPAD // CHUNK, CHUNK),
                       colp.reshape(EPAD // CHUNK, CHUNK), zrows, zhist)
    return _fin(x, s, d.reshape(NW, SROWS).T, prompt_sim_pt, prompt_degree_pt)
